# baseline (speedup 1.0000x reference)
import functools

import jax
import jax.numpy as jnp
from jax import lax
from jax.experimental import pallas as pl
from jax.experimental.pallas import tpu as pltpu
from jax.experimental.pallas import tpu_sc as plsc

_N, _E, _D = 10000, 320000, 128
_NP = 10240
_NC, _NS = 2, 16
_C = 128
_CPW = 80
_EP = _NC * _NS * _CPW * _C
_RPS = _NP // _NS
_BR = 512
_NBUF = 2


def _sc_mesh():
    return plsc.VectorSubcoreMesh(
        core_axis_name="c", subcore_axis_name="s",
        num_cores=_NC, num_subcores=_NS)


def _deg_body(dst_hbm, ones_hbm, zeros_hbm, deg_hbm, acc, didx, ones_v):
    c = lax.axis_index("c")
    s = lax.axis_index("s")
    pltpu.sync_copy(zeros_hbm.at[pl.ds(s * _RPS, _RPS)],
                    acc.at[pl.ds(s * _RPS, _RPS)])
    pltpu.sync_copy(ones_hbm, ones_v)
    row0 = (c * _NS + s) * _CPW
    pltpu.sync_copy(dst_hbm.at[pl.ds(row0, _CPW)], didx)
    plsc.subcore_barrier()

    def step(j, carry):
        pltpu.sync_copy(ones_v, acc.at[didx.at[j]], add=True)
        return carry

    lax.fori_loop(0, _CPW, step, 0)
    plsc.subcore_barrier()
    pltpu.sync_copy(acc.at[pl.ds(s * _RPS, _RPS)],
                    deg_hbm.at[pl.ds(c * _NP + s * _RPS, _RPS)])


def _deg_call(dst2d, ones_hbm, zeros1):
    return pl.kernel(
        _deg_body,
        out_type=jax.ShapeDtypeStruct((_NC * _NP,), jnp.float32),
        mesh=_sc_mesh(),
        scratch_types=[
            pltpu.VMEM_SHARED((_NP,), jnp.float32),
            pltpu.VMEM((_CPW, _C), jnp.int32),
            pltpu.VMEM((_C,), jnp.float32),
        ],
    )(dst2d, ones_hbm, zeros1)


def _prop_body(u_hbm, src_hbm, dst_hbm, out_hbm, acc, sidx, didx, rows,
               s0, s1):
    c = lax.axis_index("c")
    s = lax.axis_index("s")
    pltpu.sync_copy(u_hbm.at[pl.ds(s * _RPS, _RPS)],
                    acc.at[pl.ds(s * _RPS, _RPS)])
    row0 = (c * _NS + s) * _CPW
    pltpu.sync_copy(dst_hbm.at[pl.ds(row0, _CPW)], didx)
    plsc.subcore_barrier()
    sems = (s0, s1)

    def group(g, carry):
        k0 = g * _NBUF
        pltpu.sync_copy(src_hbm.at[pl.ds(row0 + k0, _NBUF)], sidx)
        handles = []
        for b in range(_NBUF):
            handles.append(pltpu.async_copy(
                u_hbm.at[sidx.at[b]], rows.at[b], sems[b]))
        for b in range(_NBUF):
            handles[b].wait()
            pltpu.sync_copy(rows.at[b], acc.at[didx.at[k0 + b]], add=True)
        return carry

    lax.fori_loop(0, _CPW // _NBUF, group, 0)
    plsc.subcore_barrier()
    pltpu.sync_copy(acc.at[pl.ds(s * _RPS, _RPS)],
                    out_hbm.at[pl.ds(c * _NP + s * _RPS, _RPS)])


def _prop_call(u, src2d, dst2d):
    return pl.kernel(
        _prop_body,
        out_type=jax.ShapeDtypeStruct((_NC * _NP, _D), jnp.float32),
        mesh=_sc_mesh(),
        scratch_types=[
            pltpu.VMEM_SHARED((_NP, _D), jnp.float32),
            pltpu.VMEM((_NBUF, _C), jnp.int32),
            pltpu.VMEM((_CPW, _C), jnp.int32),
            pltpu.VMEM((_NBUF, _C, _D), jnp.float32),
            pltpu.SemaphoreType.DMA,
            pltpu.SemaphoreType.DMA,
        ],
    )(u, src2d, dst2d)


def _tc0_body(deg_ref, x_ref, w_ref, u_ref):
    deg = deg_ref[:, 0:1] + deg_ref[:, 1:2]
    dis = lax.rsqrt(deg + 1.0)
    u_ref[...] = jnp.dot(x_ref[...], w_ref[...],
                         preferred_element_type=jnp.float32) * dis


def _tc0_call(deg2, xp, W0):
    grid = (_NP // _BR,)
    return pl.pallas_call(
        _tc0_body,
        grid=grid,
        in_specs=[
            pl.BlockSpec((_BR, 2), lambda i: (i, 0)),
            pl.BlockSpec((_BR, _D), lambda i: (i, 0)),
            pl.BlockSpec((_D, _D), lambda i: (0, 0)),
        ],
        out_specs=pl.BlockSpec((_BR, _D), lambda i: (i, 0)),
        out_shape=jax.ShapeDtypeStruct((_NP, _D), jnp.float32),
    )(deg2, xp, W0)


def _tcmid_body(deg_ref, s0_ref, s1_ref, up_ref, w_ref, b_ref, g_ref, be_ref,
                un_ref):
    deg = deg_ref[:, 0:1] + deg_ref[:, 1:2]
    dis = lax.rsqrt(deg + 1.0)
    z = (s0_ref[...] + s1_ref[...] - up_ref[...]) * dis + b_ref[...]
    h = jnp.maximum(z, 0.0)
    mu = jnp.mean(h, axis=-1, keepdims=True)
    d = h - mu
    var = jnp.mean(d * d, axis=-1, keepdims=True)
    hn = d * lax.rsqrt(var + 1e-5) * g_ref[...] + be_ref[...]
    un_ref[...] = jnp.dot(hn, w_ref[...],
                          preferred_element_type=jnp.float32) * dis


def _tcmid_call(deg2, sp, u_prev, Wn, bv, gv, bev):
    grid = (_NP // _BR,)
    nb = _NP // _BR
    return pl.pallas_call(
        _tcmid_body,
        grid=grid,
        in_specs=[
            pl.BlockSpec((_BR, 2), lambda i: (i, 0)),
            pl.BlockSpec((_BR, _D), lambda i: (i, 0)),
            pl.BlockSpec((_BR, _D), lambda i, nb=nb: (i + nb, 0)),
            pl.BlockSpec((_BR, _D), lambda i: (i, 0)),
            pl.BlockSpec((_D, _D), lambda i: (0, 0)),
            pl.BlockSpec((1, _D), lambda i: (0, 0)),
            pl.BlockSpec((1, _D), lambda i: (0, 0)),
            pl.BlockSpec((1, _D), lambda i: (0, 0)),
        ],
        out_specs=pl.BlockSpec((_BR, _D), lambda i: (i, 0)),
        out_shape=jax.ShapeDtypeStruct((_NP, _D), jnp.float32),
    )(deg2, sp, sp, u_prev, Wn, bv, gv, bev)


def _tcfin_body(deg_ref, s0_ref, s1_ref, up_ref, b_ref, emb_ref, h_ref):
    deg = deg_ref[:, 0:1] + deg_ref[:, 1:2]
    dis = lax.rsqrt(deg + 1.0)
    z = (s0_ref[...] + s1_ref[...] - up_ref[...]) * dis + b_ref[...]
    emb_ref[...] = z
    h_ref[...] = jnp.maximum(z, 0.0)


def _tcfin_call(deg2, sp, u_prev, bv):
    grid = (_NP // _BR,)
    nb = _NP // _BR
    return pl.pallas_call(
        _tcfin_body,
        grid=grid,
        in_specs=[
            pl.BlockSpec((_BR, 2), lambda i: (i, 0)),
            pl.BlockSpec((_BR, _D), lambda i: (i, 0)),
            pl.BlockSpec((_BR, _D), lambda i, nb=nb: (i + nb, 0)),
            pl.BlockSpec((_BR, _D), lambda i: (i, 0)),
            pl.BlockSpec((1, _D), lambda i: (0, 0)),
        ],
        out_specs=[
            pl.BlockSpec((_BR, _D), lambda i: (i, 0)),
            pl.BlockSpec((_BR, _D), lambda i: (i, 0)),
        ],
        out_shape=[
            jax.ShapeDtypeStruct((_NP, _D), jnp.float32),
            jax.ShapeDtypeStruct((_NP, _D), jnp.float32),
        ],
    )(deg2, sp, sp, u_prev, bv)


def kernel(x, edge_index, batch, W0, b0, W1, b1, W2, b2, g0, be0, g1, be1):
    f32 = jnp.float32
    xp = jnp.zeros((_NP, _D), f32).at[:_N].set(x)
    padidx = jnp.full((_EP - _E,), _NP - 1, jnp.int32)
    src2d = jnp.concatenate(
        [edge_index[0].astype(jnp.int32), padidx]).reshape(-1, _C)
    dst2d = jnp.concatenate(
        [edge_index[1].astype(jnp.int32), padidx]).reshape(-1, _C)
    zeros1 = jnp.zeros((_NP,), f32)
    ones_c = jnp.ones((_C,), f32)

    deg = _deg_call(dst2d, ones_c, zeros1)
    deg2 = deg.reshape(2, _NP).T

    b0v, b1v, b2v = (v.reshape(1, _D) for v in (b0, b1, b2))
    g0v, g1v = g0.reshape(1, _D), g1.reshape(1, _D)
    be0v, be1v = be0.reshape(1, _D), be1.reshape(1, _D)

    u0 = _tc0_call(deg2, xp, W0)
    sp0 = _prop_call(u0, src2d, dst2d)
    u1 = _tcmid_call(deg2, sp0, u0, W1, b0v, g0v, be0v)
    sp1 = _prop_call(u1, src2d, dst2d)
    u2 = _tcmid_call(deg2, sp1, u1, W2, b1v, g1v, be1v)
    sp2 = _prop_call(u2, src2d, dst2d)
    emb, h = _tcfin_call(deg2, sp2, u2, b2v)
    return emb[:_N], h[:_N]


# speedup vs baseline: 1.1321x; 1.1321x over previous
import functools

import jax
import jax.numpy as jnp
from jax import lax
from jax.experimental import pallas as pl
from jax.experimental.pallas import tpu as pltpu
from jax.experimental.pallas import tpu_sc as plsc

_N, _E, _D = 10000, 320000, 128
_NP = 10240
_NC, _NS = 2, 16
_C = 128
_CPW = 80
_EP = _NC * _NS * _CPW * _C
_RPS = _NP // _NS
_BR = 512
_NBUF = 2


def _sc_mesh():
    return plsc.VectorSubcoreMesh(
        core_axis_name="c", subcore_axis_name="s",
        num_cores=_NC, num_subcores=_NS)


def _deg_body(dst_hbm, ones_hbm, zeros_hbm, deg_hbm, acc, didx, ones_v):
    c = lax.axis_index("c")
    s = lax.axis_index("s")
    pltpu.sync_copy(zeros_hbm.at[pl.ds(s * _RPS, _RPS)],
                    acc.at[pl.ds(s * _RPS, _RPS)])
    pltpu.sync_copy(ones_hbm, ones_v)
    row0 = (c * _NS + s) * _CPW
    pltpu.sync_copy(dst_hbm.at[pl.ds(row0, _CPW)], didx)
    plsc.subcore_barrier()

    def step(j, carry):
        pltpu.sync_copy(ones_v, acc.at[didx.at[j]], add=True)
        return carry

    lax.fori_loop(0, _CPW, step, 0)
    plsc.subcore_barrier()
    pltpu.sync_copy(acc.at[pl.ds(s * _RPS, _RPS)],
                    deg_hbm.at[pl.ds(c * _NP + s * _RPS, _RPS)])


def _deg_call(dst2d, ones_hbm, zeros1):
    return pl.kernel(
        _deg_body,
        out_type=jax.ShapeDtypeStruct((_NC * _NP,), jnp.float32),
        mesh=_sc_mesh(),
        scratch_types=[
            pltpu.VMEM_SHARED((_NP,), jnp.float32),
            pltpu.VMEM((_CPW, _C), jnp.int32),
            pltpu.VMEM((_C,), jnp.float32),
        ],
    )(dst2d, ones_hbm, zeros1)


def _prop_body(u_hbm, src_hbm, dst_hbm, out_hbm, acc, sidxb, didx, rows,
               sg0, sg1, si0, si1):
    c = lax.axis_index("c")
    s = lax.axis_index("s")
    pltpu.sync_copy(u_hbm.at[pl.ds(s * _RPS, _RPS)],
                    acc.at[pl.ds(s * _RPS, _RPS)])
    row0 = (c * _NS + s) * _CPW
    pltpu.sync_copy(dst_hbm.at[pl.ds(row0, _CPW)], didx)
    pltpu.sync_copy(src_hbm.at[pl.ds(row0, 2)], sidxb.at[pl.ds(0, 2)])
    plsc.subcore_barrier()
    sg = (sg0, sg1)
    si = (si0, si1)

    pltpu.async_copy(src_hbm.at[pl.ds(row0 + 2, 2)],
                     sidxb.at[pl.ds(2, 2)], si[1])
    pltpu.async_copy(u_hbm.at[sidxb.at[0]], rows.at[0], sg[0])
    pltpu.async_copy(u_hbm.at[sidxb.at[1]], rows.at[1], sg[1])

    def dgroup(g, p, issue_next, prefetch):
        q = 1 - p
        if issue_next:
            pltpu.make_async_copy(
                src_hbm.at[pl.ds(row0 + 2 * (g + 1), 2)],
                sidxb.at[pl.ds(2 * q, 2)], si[q]).wait()
        for b in range(2):
            j = 2 * g + b
            pltpu.make_async_copy(
                u_hbm.at[sidxb.at[2 * p + b]], rows.at[b], sg[b]).wait()
            pltpu.sync_copy(rows.at[b], acc.at[didx.at[j]], add=True)
            if issue_next:
                pltpu.async_copy(
                    u_hbm.at[sidxb.at[2 * q + b]], rows.at[b], sg[b])
        if prefetch:
            pltpu.async_copy(src_hbm.at[pl.ds(row0 + 2 * (g + 2), 2)],
                             sidxb.at[pl.ds(2 * p, 2)], si[p])

    ng = _CPW // 2

    def pair(t, carry):
        dgroup(2 * t, 0, True, True)
        dgroup(2 * t + 1, 1, True, True)
        return carry

    lax.fori_loop(0, ng // 2 - 1, pair, 0)
    dgroup(ng - 2, 0, True, False)
    dgroup(ng - 1, 1, False, False)
    plsc.subcore_barrier()
    pltpu.sync_copy(acc.at[pl.ds(s * _RPS, _RPS)],
                    out_hbm.at[pl.ds(c * _NP + s * _RPS, _RPS)])


def _prop_call(u, src2d, dst2d):
    return pl.kernel(
        _prop_body,
        out_type=jax.ShapeDtypeStruct((_NC * _NP, _D), jnp.float32),
        mesh=_sc_mesh(),
        scratch_types=[
            pltpu.VMEM_SHARED((_NP, _D), jnp.float32),
            pltpu.VMEM((4, _C), jnp.int32),
            pltpu.VMEM((_CPW, _C), jnp.int32),
            pltpu.VMEM((2, _C, _D), jnp.float32),
            pltpu.SemaphoreType.DMA,
            pltpu.SemaphoreType.DMA,
            pltpu.SemaphoreType.DMA,
            pltpu.SemaphoreType.DMA,
        ],
    )(u, src2d, dst2d)


def _tc0_body(deg_ref, x_ref, w_ref, u_ref):
    deg = deg_ref[:, 0:1] + deg_ref[:, 1:2]
    dis = lax.rsqrt(deg + 1.0)
    u_ref[...] = jnp.dot(x_ref[...], w_ref[...],
                         preferred_element_type=jnp.float32) * dis


def _tc0_call(deg2, xp, W0):
    grid = (_NP // _BR,)
    return pl.pallas_call(
        _tc0_body,
        grid=grid,
        in_specs=[
            pl.BlockSpec((_BR, 2), lambda i: (i, 0)),
            pl.BlockSpec((_BR, _D), lambda i: (i, 0)),
            pl.BlockSpec((_D, _D), lambda i: (0, 0)),
        ],
        out_specs=pl.BlockSpec((_BR, _D), lambda i: (i, 0)),
        out_shape=jax.ShapeDtypeStruct((_NP, _D), jnp.float32),
    )(deg2, xp, W0)


def _tcmid_body(deg_ref, s0_ref, s1_ref, up_ref, w_ref, b_ref, g_ref, be_ref,
                un_ref):
    deg = deg_ref[:, 0:1] + deg_ref[:, 1:2]
    dis = lax.rsqrt(deg + 1.0)
    z = (s0_ref[...] + s1_ref[...] - up_ref[...]) * dis + b_ref[...]
    h = jnp.maximum(z, 0.0)
    mu = jnp.mean(h, axis=-1, keepdims=True)
    d = h - mu
    var = jnp.mean(d * d, axis=-1, keepdims=True)
    hn = d * lax.rsqrt(var + 1e-5) * g_ref[...] + be_ref[...]
    un_ref[...] = jnp.dot(hn, w_ref[...],
                          preferred_element_type=jnp.float32) * dis


def _tcmid_call(deg2, sp, u_prev, Wn, bv, gv, bev):
    grid = (_NP // _BR,)
    nb = _NP // _BR
    return pl.pallas_call(
        _tcmid_body,
        grid=grid,
        in_specs=[
            pl.BlockSpec((_BR, 2), lambda i: (i, 0)),
            pl.BlockSpec((_BR, _D), lambda i: (i, 0)),
            pl.BlockSpec((_BR, _D), lambda i, nb=nb: (i + nb, 0)),
            pl.BlockSpec((_BR, _D), lambda i: (i, 0)),
            pl.BlockSpec((_D, _D), lambda i: (0, 0)),
            pl.BlockSpec((1, _D), lambda i: (0, 0)),
            pl.BlockSpec((1, _D), lambda i: (0, 0)),
            pl.BlockSpec((1, _D), lambda i: (0, 0)),
        ],
        out_specs=pl.BlockSpec((_BR, _D), lambda i: (i, 0)),
        out_shape=jax.ShapeDtypeStruct((_NP, _D), jnp.float32),
    )(deg2, sp, sp, u_prev, Wn, bv, gv, bev)


def _tcfin_body(deg_ref, s0_ref, s1_ref, up_ref, b_ref, emb_ref, h_ref):
    deg = deg_ref[:, 0:1] + deg_ref[:, 1:2]
    dis = lax.rsqrt(deg + 1.0)
    z = (s0_ref[...] + s1_ref[...] - up_ref[...]) * dis + b_ref[...]
    emb_ref[...] = z
    h_ref[...] = jnp.maximum(z, 0.0)


def _tcfin_call(deg2, sp, u_prev, bv):
    grid = (_NP // _BR,)
    nb = _NP // _BR
    return pl.pallas_call(
        _tcfin_body,
        grid=grid,
        in_specs=[
            pl.BlockSpec((_BR, 2), lambda i: (i, 0)),
            pl.BlockSpec((_BR, _D), lambda i: (i, 0)),
            pl.BlockSpec((_BR, _D), lambda i, nb=nb: (i + nb, 0)),
            pl.BlockSpec((_BR, _D), lambda i: (i, 0)),
            pl.BlockSpec((1, _D), lambda i: (0, 0)),
        ],
        out_specs=[
            pl.BlockSpec((_BR, _D), lambda i: (i, 0)),
            pl.BlockSpec((_BR, _D), lambda i: (i, 0)),
        ],
        out_shape=[
            jax.ShapeDtypeStruct((_NP, _D), jnp.float32),
            jax.ShapeDtypeStruct((_NP, _D), jnp.float32),
        ],
    )(deg2, sp, sp, u_prev, bv)


def kernel(x, edge_index, batch, W0, b0, W1, b1, W2, b2, g0, be0, g1, be1):
    f32 = jnp.float32
    xp = jnp.zeros((_NP, _D), f32).at[:_N].set(x)
    padidx = jnp.full((_EP - _E,), _NP - 1, jnp.int32)
    src2d = jnp.concatenate(
        [edge_index[0].astype(jnp.int32), padidx]).reshape(-1, _C)
    dst2d = jnp.concatenate(
        [edge_index[1].astype(jnp.int32), padidx]).reshape(-1, _C)
    zeros1 = jnp.zeros((_NP,), f32)
    ones_c = jnp.ones((_C,), f32)

    deg = _deg_call(dst2d, ones_c, zeros1)
    deg2 = deg.reshape(2, _NP).T

    b0v, b1v, b2v = (v.reshape(1, _D) for v in (b0, b1, b2))
    g0v, g1v = g0.reshape(1, _D), g1.reshape(1, _D)
    be0v, be1v = be0.reshape(1, _D), be1.reshape(1, _D)

    u0 = _tc0_call(deg2, xp, W0)
    sp0 = _prop_call(u0, src2d, dst2d)
    u1 = _tcmid_call(deg2, sp0, u0, W1, b0v, g0v, be0v)
    sp1 = _prop_call(u1, src2d, dst2d)
    u2 = _tcmid_call(deg2, sp1, u1, W2, b1v, g1v, be1v)
    sp2 = _prop_call(u2, src2d, dst2d)
    emb, h = _tcfin_call(deg2, sp2, u2, b2v)
    return emb[:_N], h[:_N]


# speedup vs baseline: 2.1408x; 1.8909x over previous
import functools

import jax
import jax.numpy as jnp
from jax import lax
from jax.experimental import pallas as pl
from jax.experimental.pallas import tpu as pltpu
from jax.experimental.pallas import tpu_sc as plsc

_N, _E, _D = 10000, 320000, 128
_NP = 10240
_NC, _NS = 2, 16
_C = 112
_BS = 6
_NBK = 15
_CPW = _BS * _NBK
_EP = _NC * _NS * _CPW * _C
_RPS = _NP // _NS
_BR = 512
_NBUF = 2


def _sc_mesh():
    return plsc.VectorSubcoreMesh(
        core_axis_name="c", subcore_axis_name="s",
        num_cores=_NC, num_subcores=_NS)


def _deg_body(dst_hbm, ones_hbm, zeros_hbm, deg_hbm, acc, didx, ones_v):
    c = lax.axis_index("c")
    s = lax.axis_index("s")
    pltpu.sync_copy(zeros_hbm.at[pl.ds(s * _RPS, _RPS)],
                    acc.at[pl.ds(s * _RPS, _RPS)])
    pltpu.sync_copy(ones_hbm, ones_v)
    blk0 = (c * _NS + s) * _NBK
    pltpu.sync_copy(dst_hbm.at[pl.ds(blk0, _NBK)], didx)
    plsc.subcore_barrier()

    def step(blk, carry):
        for u in range(_BS):
            pltpu.sync_copy(ones_v, acc.at[didx.at[blk, u]], add=True)
        return carry

    lax.fori_loop(0, _NBK, step, 0)
    plsc.subcore_barrier()
    pltpu.sync_copy(acc.at[pl.ds(s * _RPS, _RPS)],
                    deg_hbm.at[pl.ds(c * _NP + s * _RPS, _RPS)])


def _deg_call(dst2d, ones_hbm, zeros1):
    return pl.kernel(
        _deg_body,
        out_type=jax.ShapeDtypeStruct((_NC * _NP,), jnp.float32),
        mesh=_sc_mesh(),
        scratch_types=[
            pltpu.VMEM_SHARED((_NP,), jnp.float32),
            pltpu.VMEM((_NBK, _BS, _C), jnp.int32),
            pltpu.VMEM((_C,), jnp.float32),
        ],
    )(dst2d, ones_hbm, zeros1)


def _prop_body(u_hbm, src_hbm, dst_hbm, out_hbm, acc, sidxb, didxb, rows,
               sg0, sg1, sg2, ss0, ss1, ss2, si0, si1, di0, di1):
    c = lax.axis_index("c")
    s = lax.axis_index("s")
    pltpu.sync_copy(u_hbm.at[pl.ds(s * _RPS, _RPS)],
                    acc.at[pl.ds(s * _RPS, _RPS)])
    blk0 = (c * _NS + s) * _NBK
    pltpu.sync_copy(src_hbm.at[blk0], sidxb.at[0])
    pltpu.sync_copy(dst_hbm.at[blk0], didxb.at[0])
    plsc.subcore_barrier()
    sg = (sg0, sg1, sg2)
    ss = (ss0, ss1, ss2)
    si = (si0, si1)
    di = (di0, di1)

    pltpu.async_copy(u_hbm.at[pl.ds(0, _C)], rows.at[1], ss[1])
    pltpu.async_copy(u_hbm.at[pl.ds(0, _C)], rows.at[2], ss[2])
    pltpu.async_copy(u_hbm.at[sidxb.at[0, 0]], rows.at[0], sg[0])

    def swait(b, idx_row):
        pltpu.make_async_copy(rows.at[b], acc.at[idx_row], ss[b]).wait()

    def block(S, P, last):
        Q = 1 - P
        for u in range(_BS):
            b = u % 3
            bn = (u + 1) % 3
            if u == 2 and not last:
                pltpu.async_copy(src_hbm.at[blk0 + S + 1], sidxb.at[Q],
                                 si[Q])
                pltpu.async_copy(dst_hbm.at[blk0 + S + 1], didxb.at[Q],
                                 di[Q])
            if u < _BS - 1:
                swait(bn, didxb.at[P, (u + 4) % _BS])
                pltpu.async_copy(u_hbm.at[sidxb.at[P, u + 1]], rows.at[bn],
                                 sg[bn])
            elif not last:
                pltpu.make_async_copy(src_hbm.at[blk0 + S + 1],
                                      sidxb.at[Q], si[Q]).wait()
                pltpu.make_async_copy(dst_hbm.at[blk0 + S + 1],
                                      didxb.at[Q], di[Q]).wait()
                swait(bn, didxb.at[P, _BS - 2])
                pltpu.async_copy(u_hbm.at[sidxb.at[Q, 0]], rows.at[bn],
                                 sg[bn])
            pltpu.make_async_copy(u_hbm.at[sidxb.at[P, u]], rows.at[b],
                                  sg[b]).wait()
            pltpu.async_copy(rows.at[b], acc.at[didxb.at[P, u]], ss[b],
                             add=True)

    def pair(t, carry):
        block(2 * t, 0, False)
        block(2 * t + 1, 1, False)
        return carry

    lax.fori_loop(0, (_NBK - 1) // 2, pair, 0)
    block(_NBK - 1, (_NBK - 1) % 2, True)
    for b in range(3):
        swait(b, didxb.at[(_NBK - 1) % 2, b])
    plsc.subcore_barrier()
    pltpu.sync_copy(acc.at[pl.ds(s * _RPS, _RPS)],
                    out_hbm.at[pl.ds(c * _NP + s * _RPS, _RPS)])


def _prop_call(u, src2d, dst2d):
    return pl.kernel(
        _prop_body,
        out_type=jax.ShapeDtypeStruct((_NC * _NP, _D), jnp.float32),
        mesh=_sc_mesh(),
        scratch_types=[
            pltpu.VMEM_SHARED((_NP, _D), jnp.float32),
            pltpu.VMEM((2, _BS, _C), jnp.int32),
            pltpu.VMEM((2, _BS, _C), jnp.int32),
            pltpu.VMEM((3, _C, _D), jnp.float32),
            pltpu.SemaphoreType.DMA,
            pltpu.SemaphoreType.DMA,
            pltpu.SemaphoreType.DMA,
            pltpu.SemaphoreType.DMA,
            pltpu.SemaphoreType.DMA,
            pltpu.SemaphoreType.DMA,
            pltpu.SemaphoreType.DMA,
            pltpu.SemaphoreType.DMA,
            pltpu.SemaphoreType.DMA,
            pltpu.SemaphoreType.DMA,
        ],
    )(u, src2d, dst2d)


def _tc0_body(deg_ref, x_ref, w_ref, u_ref):
    deg = deg_ref[:, 0:1] + deg_ref[:, 1:2]
    dis = lax.rsqrt(deg + 1.0)
    u_ref[...] = jnp.dot(x_ref[...], w_ref[...],
                         preferred_element_type=jnp.float32) * dis


def _tc0_call(deg2, xp, W0):
    grid = (_NP // _BR,)
    return pl.pallas_call(
        _tc0_body,
        grid=grid,
        in_specs=[
            pl.BlockSpec((_BR, 2), lambda i: (i, 0)),
            pl.BlockSpec((_BR, _D), lambda i: (i, 0)),
            pl.BlockSpec((_D, _D), lambda i: (0, 0)),
        ],
        out_specs=pl.BlockSpec((_BR, _D), lambda i: (i, 0)),
        out_shape=jax.ShapeDtypeStruct((_NP, _D), jnp.float32),
    )(deg2, xp, W0)


def _tcmid_body(deg_ref, s0_ref, s1_ref, up_ref, w_ref, b_ref, g_ref, be_ref,
                un_ref):
    deg = deg_ref[:, 0:1] + deg_ref[:, 1:2]
    dis = lax.rsqrt(deg + 1.0)
    z = (s0_ref[...] + s1_ref[...] - up_ref[...]) * dis + b_ref[...]
    h = jnp.maximum(z, 0.0)
    mu = jnp.mean(h, axis=-1, keepdims=True)
    d = h - mu
    var = jnp.mean(d * d, axis=-1, keepdims=True)
    hn = d * lax.rsqrt(var + 1e-5) * g_ref[...] + be_ref[...]
    un_ref[...] = jnp.dot(hn, w_ref[...],
                          preferred_element_type=jnp.float32) * dis


def _tcmid_call(deg2, sp, u_prev, Wn, bv, gv, bev):
    grid = (_NP // _BR,)
    nb = _NP // _BR
    return pl.pallas_call(
        _tcmid_body,
        grid=grid,
        in_specs=[
            pl.BlockSpec((_BR, 2), lambda i: (i, 0)),
            pl.BlockSpec((_BR, _D), lambda i: (i, 0)),
            pl.BlockSpec((_BR, _D), lambda i, nb=nb: (i + nb, 0)),
            pl.BlockSpec((_BR, _D), lambda i: (i, 0)),
            pl.BlockSpec((_D, _D), lambda i: (0, 0)),
            pl.BlockSpec((1, _D), lambda i: (0, 0)),
            pl.BlockSpec((1, _D), lambda i: (0, 0)),
            pl.BlockSpec((1, _D), lambda i: (0, 0)),
        ],
        out_specs=pl.BlockSpec((_BR, _D), lambda i: (i, 0)),
        out_shape=jax.ShapeDtypeStruct((_NP, _D), jnp.float32),
    )(deg2, sp, sp, u_prev, Wn, bv, gv, bev)


def _tcfin_body(deg_ref, s0_ref, s1_ref, up_ref, b_ref, emb_ref, h_ref):
    deg = deg_ref[:, 0:1] + deg_ref[:, 1:2]
    dis = lax.rsqrt(deg + 1.0)
    z = (s0_ref[...] + s1_ref[...] - up_ref[...]) * dis + b_ref[...]
    emb_ref[...] = z
    h_ref[...] = jnp.maximum(z, 0.0)


def _tcfin_call(deg2, sp, u_prev, bv):
    grid = (_NP // _BR,)
    nb = _NP // _BR
    return pl.pallas_call(
        _tcfin_body,
        grid=grid,
        in_specs=[
            pl.BlockSpec((_BR, 2), lambda i: (i, 0)),
            pl.BlockSpec((_BR, _D), lambda i: (i, 0)),
            pl.BlockSpec((_BR, _D), lambda i, nb=nb: (i + nb, 0)),
            pl.BlockSpec((_BR, _D), lambda i: (i, 0)),
            pl.BlockSpec((1, _D), lambda i: (0, 0)),
        ],
        out_specs=[
            pl.BlockSpec((_BR, _D), lambda i: (i, 0)),
            pl.BlockSpec((_BR, _D), lambda i: (i, 0)),
        ],
        out_shape=[
            jax.ShapeDtypeStruct((_NP, _D), jnp.float32),
            jax.ShapeDtypeStruct((_NP, _D), jnp.float32),
        ],
    )(deg2, sp, sp, u_prev, bv)


def kernel(x, edge_index, batch, W0, b0, W1, b1, W2, b2, g0, be0, g1, be1):
    f32 = jnp.float32
    xp = jnp.zeros((_NP, _D), f32).at[:_N].set(x)
    padidx = jnp.full((_EP - _E,), _NP - 1, jnp.int32)
    src2d = jnp.concatenate(
        [edge_index[0].astype(jnp.int32), padidx]).reshape(-1, _BS, _C)
    dst2d = jnp.concatenate(
        [edge_index[1].astype(jnp.int32), padidx]).reshape(-1, _BS, _C)
    zeros1 = jnp.zeros((_NP,), f32)
    ones_c = jnp.ones((_C,), f32)

    deg = _deg_call(dst2d, ones_c, zeros1)
    deg2 = deg.reshape(2, _NP).T

    b0v, b1v, b2v = (v.reshape(1, _D) for v in (b0, b1, b2))
    g0v, g1v = g0.reshape(1, _D), g1.reshape(1, _D)
    be0v, be1v = be0.reshape(1, _D), be1.reshape(1, _D)

    u0 = _tc0_call(deg2, xp, W0)
    sp0 = _prop_call(u0, src2d, dst2d)
    u1 = _tcmid_call(deg2, sp0, u0, W1, b0v, g0v, be0v)
    sp1 = _prop_call(u1, src2d, dst2d)
    u2 = _tcmid_call(deg2, sp1, u1, W2, b1v, g1v, be1v)
    sp2 = _prop_call(u2, src2d, dst2d)
    emb, h = _tcfin_call(deg2, sp2, u2, b2v)
    return emb[:_N], h[:_N]
